# Initial kernel scaffold; baseline (speedup 1.0000x reference)
import jax
import jax.numpy as jnp
from jax.experimental import pallas as pl


def kernel(input_ids_0, input_ids_1, attention_type_ids, relative_position_ids, table_0, table_1, attn_type_table, inverse_freqs):
    raise NotImplementedError("write your pallas kernel here")


# speedup vs baseline: 1.9366x; 1.9366x over previous
import functools

import jax
import jax.numpy as jnp
from jax import lax
from jax.experimental import pallas as pl
from jax.experimental.pallas import tpu as pltpu
from jax.experimental.pallas import tpu_sc as plsc


def _emb_sum_call(table_0, table_1, ids0, ids1):
    H = table_0.shape[1]
    N = ids0.shape[0]
    info = plsc.get_sparse_core_info()
    ncores, nsub, nlanes = info.num_cores, info.num_subcores, info.num_lanes
    NW = ncores * nsub
    NPW = N // NW
    C = nlanes
    NCH = NPW // C
    NBUF = 2
    HV = H // nlanes

    mesh = plsc.VectorSubcoreMesh(core_axis_name="c", subcore_axis_name="s")

    @functools.partial(
        pl.kernel,
        mesh=mesh,
        out_type=jax.ShapeDtypeStruct((N, H), jnp.float32),
        scratch_types=[
            pltpu.VMEM((NPW,), jnp.int32),
            pltpu.VMEM((NPW,), jnp.int32),
            pltpu.VMEM((NBUF, C, H), jnp.float32),
            pltpu.VMEM((NBUF, C, H), jnp.float32),
            pltpu.VMEM((NBUF, C, H), jnp.float32),
            pltpu.SemaphoreType.DMA,
            pltpu.SemaphoreType.DMA,
            pltpu.SemaphoreType.DMA,
            pltpu.SemaphoreType.DMA,
            pltpu.SemaphoreType.DMA,
            pltpu.SemaphoreType.DMA,
        ],
    )
    def emb(t0, t1, i0, i1, out, idx0_v, idx1_v, buf_a, buf_b, obuf,
            sem_a0, sem_a1, sem_b0, sem_b1, sem_s0, sem_s1):
        sems_a = (sem_a0, sem_a1)
        sems_b = (sem_b0, sem_b1)
        sems_s = (sem_s0, sem_s1)
        wid = lax.axis_index("s") * ncores + lax.axis_index("c")
        base = wid * NPW
        pltpu.sync_copy(i0.at[pl.ds(base, NPW)], idx0_v)
        pltpu.sync_copy(i1.at[pl.ds(base, NPW)], idx1_v)

        def issue_gathers(i, b):
            iv0 = idx0_v[pl.ds(i * C, C)]
            iv1 = idx1_v[pl.ds(i * C, C)]
            pltpu.async_copy(t0.at[iv0], buf_a.at[b], sems_a[b])
            pltpu.async_copy(t1.at[iv1], buf_b.at[b], sems_b[b])

        for b in range(NBUF):
            issue_gathers(b, b)

        def outer(it, carry):
            g = it * NBUF
            for b in range(NBUF):
                i = g + b
                pltpu.make_async_copy(
                    t0.at[pl.ds(0, C)], buf_a.at[b], sems_a[b]).wait()
                pltpu.make_async_copy(
                    t1.at[pl.ds(0, C)], buf_b.at[b], sems_b[b]).wait()

                @pl.when(i >= NBUF)
                def _():
                    pltpu.make_async_copy(
                        obuf.at[b], out.at[pl.ds(0, C)], sems_s[b]).wait()

                def addrow(r, c2):
                    for cc in range(HV):
                        sl = pl.ds(cc * nlanes, nlanes)
                        obuf[b, r, sl] = buf_a[b, r, sl] + buf_b[b, r, sl]
                    return c2
                lax.fori_loop(0, C, addrow, 0)

                @pl.when(i + NBUF < NCH)
                def _():
                    issue_gathers(i + NBUF, b)

                pltpu.async_copy(
                    obuf.at[b], out.at[pl.ds(base + i * C, C)], sems_s[b])
            return carry

        lax.fori_loop(0, NCH // NBUF, outer, 0)

        for b in range(NBUF):
            pltpu.make_async_copy(
                obuf.at[b], out.at[pl.ds(0, C)], sems_s[b]).wait()

    return emb(table_0, table_1, ids0, ids1)


def _pos_att_call(rel, typ, invf2, att_table):
    N = rel.shape[0]
    Hh = att_table.shape[1]
    T = att_table.shape[0]
    R = 512
    G = N // R

    def body(rel_ref, typ_ref, invf_ref, tab_ref, out_ref):
        relv = rel_ref[...].astype(jnp.float32)
        x = relv[:, None] / invf_ref[...][None, :]
        col = lax.broadcasted_iota(jnp.int32, (R, Hh), 1)
        pe = jnp.where(col < Hh // 2, jnp.sin(x), jnp.cos(x))
        onehot = (typ_ref[...][:, None]
                  == lax.broadcasted_iota(jnp.int32, (R, T), 1))
        att = jnp.dot(onehot.astype(jnp.float32), tab_ref[...],
                      preferred_element_type=jnp.float32)
        out_ref[...] = pe + att

    return pl.pallas_call(
        body,
        grid=(G,),
        in_specs=[
            pl.BlockSpec((R,), lambda i: (i,)),
            pl.BlockSpec((R,), lambda i: (i,)),
            pl.BlockSpec((Hh,), lambda i: (0,)),
            pl.BlockSpec((T, Hh), lambda i: (0, 0)),
        ],
        out_specs=pl.BlockSpec((R, Hh), lambda i: (i, 0)),
        out_shape=jax.ShapeDtypeStruct((N, Hh), jnp.float32),
    )(rel, typ, invf2, att_table)


def kernel(input_ids_0, input_ids_1, attention_type_ids,
           relative_position_ids, table_0, table_1, attn_type_table,
           inverse_freqs):
    B, L = input_ids_0.shape
    H = table_0.shape[1]
    Hh = attn_type_table.shape[1]
    out1 = _emb_sum_call(
        table_0, table_1,
        input_ids_0.reshape(-1), input_ids_1.reshape(-1)).reshape(B, L, H)
    invf2 = jnp.concatenate([inverse_freqs, inverse_freqs])
    out2 = _pos_att_call(
        relative_position_ids.reshape(-1), attention_type_ids.reshape(-1),
        invf2, attn_type_table).reshape(B, L, Hh)
    return (out1, out2)
